# baseline (speedup 1.0000x reference)
import functools

import jax
import jax.numpy as jnp
from jax import lax
from jax.experimental import pallas as pl
from jax.experimental.pallas import tpu as pltpu
from jax.experimental.pallas import tpu_sc as plsc

N_NODES = 100000
N_REL = 3
N_EDGES = 200000
FEAT = 128
LANES = 16
CSLICES = FEAT // LANES
N_PAD = 100096
E_PAD = 200704
N_TILES = 16
N_CORES = 2
STRIPE = N_PAD // N_TILES
EDGES_PER_TILE = E_PAD // N_TILES
BATCH = 128
N_BATCH = EDGES_PER_TILE // BATCH
ZROWS = 782
N_PASS = N_REL * CSLICES + N_REL
PASS_PER_CORE = 14


def _sc_segment_sums(xt_flat, src, dst, zeros_in, ones_in):
  mesh = plsc.VectorSubcoreMesh(core_axis_name="c", subcore_axis_name="s")

  @functools.partial(
      pl.kernel,
      mesh=mesh,
      compiler_params=pltpu.CompilerParams(use_tc_tiling_on_sc=False),
      out_type=[
          jax.ShapeDtypeStruct((N_REL, CSLICES, N_PAD, LANES), jnp.float32),
          jax.ShapeDtypeStruct((N_REL, N_PAD, LANES), jnp.float32),
      ],
      scratch_types=[
          pltpu.VMEM((ZROWS, LANES), jnp.float32),
          pltpu.VMEM((BATCH, LANES), jnp.float32),
          pltpu.VMEM((BATCH, LANES), jnp.float32),
          pltpu.VMEM((BATCH,), jnp.int32),
          pltpu.VMEM((BATCH,), jnp.int32),
          pltpu.VMEM_SHARED((N_PAD, LANES), jnp.float32),
          pltpu.SemaphoreType.DMA,
      ],
  )
  def k(xt_hbm, src_hbm, dst_hbm, zin_hbm, oin_hbm,
        agg_hbm, deg_hbm,
        zerobuf, onesbuf, rows, sidx, didx, acc, sem):
    cid = lax.axis_index("c")
    tid = lax.axis_index("s")
    ebase0 = tid * EDGES_PER_TILE
    rbase = tid * STRIPE

    pltpu.sync_copy(zin_hbm, zerobuf)
    pltpu.sync_copy(oin_hbm, onesbuf)

    def one_pass(j, _):
      pid = 2 * j + cid

      @pl.when(pid < N_PASS)
      def _run():
        for z in range(STRIPE // ZROWS):
          pltpu.sync_copy(zerobuf, acc.at[pl.ds(rbase + z * ZROWS, ZROWS)])
        plsc.subcore_barrier()

        @pl.when(pid < N_REL * CSLICES)
        def _feature_pass():
          r = pid // CSLICES
          c = pid % CSLICES
          coff = c * N_NODES

          def batch(i, _):
            ebase = ebase0 + i * BATCH
            pltpu.sync_copy(src_hbm.at[r, pl.ds(ebase, BATCH)], sidx)
            pltpu.sync_copy(dst_hbm.at[r, pl.ds(ebase, BATCH)], didx)
            for kk in range(BATCH // LANES):
              sl = pl.ds(kk * LANES, LANES)
              sidx[sl] = sidx[sl] + coff
            pltpu.async_copy(xt_hbm.at[sidx], rows, sem).wait()
            pltpu.sync_copy(rows, acc.at[didx], add=True)
            return 0

          lax.fori_loop(0, N_BATCH, batch, 0)
          plsc.subcore_barrier()
          pltpu.sync_copy(acc.at[pl.ds(rbase, STRIPE)],
                          agg_hbm.at[r, c, pl.ds(rbase, STRIPE)])

        @pl.when(pid >= N_REL * CSLICES)
        def _degree_pass():
          r = pid - N_REL * CSLICES

          def batch(i, _):
            ebase = ebase0 + i * BATCH
            pltpu.sync_copy(dst_hbm.at[r, pl.ds(ebase, BATCH)], didx)
            pltpu.sync_copy(onesbuf, acc.at[didx], add=True)
            return 0

          lax.fori_loop(0, N_BATCH, batch, 0)
          plsc.subcore_barrier()
          pltpu.sync_copy(acc.at[pl.ds(rbase, STRIPE)],
                          deg_hbm.at[r, pl.ds(rbase, STRIPE)])

      return 0

    lax.fori_loop(0, PASS_PER_CORE, one_pass, 0)

  return k(xt_flat, src, dst, zeros_in, ones_in)


def _tc_combine(agg, deg, W, b3):
  BN = 256
  grid = (N_PAD // BN,)

  def body(agg_ref, deg_ref, w_ref, b_ref, out_ref):
    acc = (b_ref[0, 0] + b_ref[1, 0] + b_ref[2, 0])[None, :]
    acc = jnp.broadcast_to(acc, (BN, FEAT))
    for r in range(N_REL):
      agg_r = jnp.concatenate([agg_ref[r, c] for c in range(CSLICES)], axis=1)
      h = jax.lax.dot_general(
          agg_r, w_ref[r], (((1,), (0,)), ((), ())),
          precision=jax.lax.Precision.HIGHEST,
          preferred_element_type=jnp.float32)
      d = deg_ref[r, :, 0]
      norm = jnp.where(d > 0.0, 1.0 / d, 0.0)
      acc = acc + h * norm[:, None]
    out_ref[...] = acc

  return pl.pallas_call(
      body,
      grid=grid,
      in_specs=[
          pl.BlockSpec((N_REL, CSLICES, BN, LANES), lambda i: (0, 0, i, 0)),
          pl.BlockSpec((N_REL, BN, LANES), lambda i: (0, i, 0)),
          pl.BlockSpec((N_REL, FEAT, FEAT), lambda i: (0, 0, 0)),
          pl.BlockSpec((N_REL, 1, FEAT), lambda i: (0, 0, 0)),
      ],
      out_specs=pl.BlockSpec((BN, FEAT), lambda i: (i, 0)),
      out_shape=jax.ShapeDtypeStruct((N_PAD, FEAT), jnp.float32),
  )(agg, deg, W, b3)


def kernel(x, edge_index, W, b):
  src = edge_index[:, 0, :].astype(jnp.int32)
  dst = edge_index[:, 1, :].astype(jnp.int32)
  pad = E_PAD - N_EDGES
  src = jnp.pad(src, ((0, 0), (0, pad)))
  dst = jnp.pad(dst, ((0, 0), (0, pad)), constant_values=N_NODES)

  xt_flat = x.reshape(N_NODES, CSLICES, LANES).transpose(1, 0, 2)
  xt_flat = xt_flat.reshape(CSLICES * N_NODES, LANES)

  zeros_in = jnp.zeros((ZROWS, LANES), jnp.float32)
  ones_in = jnp.ones((BATCH, LANES), jnp.float32)

  agg, deg = _sc_segment_sums(xt_flat, src, dst, zeros_in, ones_in)
  out = _tc_combine(agg, deg, W, b.reshape(N_REL, 1, FEAT))
  return out[:N_NODES]


# speedup vs baseline: 2.5345x; 2.5345x over previous
import functools

import jax
import jax.numpy as jnp
from jax import lax
from jax.experimental import pallas as pl
from jax.experimental.pallas import tpu as pltpu
from jax.experimental.pallas import tpu_sc as plsc

N_NODES = 100000
N_REL = 3
N_EDGES = 200000
FEAT = 128
LANES = 16
CSLICES = FEAT // LANES
N_PAD = 100096
E_PAD = 200704
N_TILES = 16
STRIPE = N_PAD // N_TILES
EDGES_PER_TILE = E_PAD // N_TILES
BATCH = 128
N_BATCH = EDGES_PER_TILE // BATCH
SEG_B = 14
N_SEG = N_BATCH // SEG_B
ZROWS = 391
N_PASS = N_REL * CSLICES + N_REL
PASS_PER_CORE = 14


def _sc_segment_sums(xflat, sidx, dst4, zeros_in, ones_in):
  mesh = plsc.VectorSubcoreMesh(core_axis_name="c", subcore_axis_name="s")

  @functools.partial(
      pl.kernel,
      mesh=mesh,
      compiler_params=pltpu.CompilerParams(use_tc_tiling_on_sc=False),
      out_type=[
          jax.ShapeDtypeStruct((N_REL, N_PAD, FEAT), jnp.float32),
          jax.ShapeDtypeStruct((N_REL, N_PAD, LANES), jnp.float32),
      ],
      scratch_types=[
          pltpu.VMEM((ZROWS, LANES), jnp.float32),
          pltpu.VMEM((BATCH, LANES), jnp.float32),
          pltpu.VMEM((BATCH, LANES), jnp.float32),
          pltpu.VMEM((BATCH, LANES), jnp.float32),
          pltpu.VMEM((SEG_B * BATCH,), jnp.int32),
          pltpu.VMEM((SEG_B, BATCH), jnp.int32),
          pltpu.VMEM_SHARED((N_PAD, LANES), jnp.float32),
          pltpu.SemaphoreType.DMA,
          pltpu.SemaphoreType.DMA,
          pltpu.SemaphoreType.DMA,
          pltpu.SemaphoreType.DMA,
      ],
  )
  def k(xflat_hbm, sidx_hbm, dst_hbm, zin_hbm, oin_hbm,
        agg_hbm, deg_hbm,
        zerobuf, onesbuf, rows_a, rows_b, srcbuf, dstbuf, acc,
        gsem_a, gsem_b, ssem_a, ssem_b):
    cid = lax.axis_index("c")
    tid = lax.axis_index("s")
    rbase = tid * STRIPE

    pltpu.sync_copy(zin_hbm, zerobuf)
    pltpu.sync_copy(oin_hbm, onesbuf)

    def gather_desc(i, buf, sem):
      return pltpu.make_async_copy(
          xflat_hbm.at[srcbuf.at[pl.ds(i * BATCH, BATCH)]], buf, sem)

    def scat_desc(rows, i, sem):
      return pltpu.make_async_copy(rows, acc.at[dstbuf.at[i]], sem)

    def zero_stripe():
      for z in range(STRIPE // ZROWS):
        pltpu.async_copy(zerobuf, acc.at[pl.ds(rbase + z * ZROWS, ZROWS)],
                         ssem_a)
      for _ in range(STRIPE // ZROWS):
        pltpu.make_async_copy(
            zerobuf, acc.at[pl.ds(rbase, ZROWS)], ssem_a).wait()

    def one_pass(j, _):
      pid = 2 * j + cid

      @pl.when(pid < N_PASS)
      def _run():
        @pl.when(pid < N_REL * CSLICES)
        def _feature_pass():
          r = pid // CSLICES
          c = pid % CSLICES
          zero_stripe()
          plsc.subcore_barrier()

          def segment(s, _):
            pltpu.sync_copy(sidx_hbm.at[r, c, tid, s], srcbuf)
            pltpu.sync_copy(dst_hbm.at[r, tid, s], dstbuf)
            gather_desc(0, rows_a, gsem_a).start()

            def pair(p, _):
              i0 = 2 * p
              i1 = 2 * p + 1

              @pl.when(p > 0)
              def _():
                scat_desc(rows_b, i0 - 1, ssem_b).wait()

              gather_desc(i1, rows_b, gsem_b).start()
              gather_desc(i0, rows_a, gsem_a).wait()
              scat_desc(rows_a, i0, ssem_a).start(add=True)
              scat_desc(rows_a, i0, ssem_a).wait()

              @pl.when(i1 + 1 < SEG_B)
              def _():
                gather_desc(i1 + 1, rows_a, gsem_a).start()

              gather_desc(i1, rows_b, gsem_b).wait()
              scat_desc(rows_b, i1, ssem_b).start(add=True)
              return 0

            lax.fori_loop(0, SEG_B // 2, pair, 0)
            scat_desc(rows_b, SEG_B - 1, ssem_b).wait()
            return 0

          lax.fori_loop(0, N_SEG, segment, 0)
          plsc.subcore_barrier()
          pltpu.sync_copy(
              acc.at[pl.ds(rbase, STRIPE)],
              agg_hbm.at[r, pl.ds(rbase, STRIPE), pl.ds(c * LANES, LANES)])

        @pl.when(pid >= N_REL * CSLICES)
        def _degree_pass():
          r = pid - N_REL * CSLICES
          zero_stripe()
          plsc.subcore_barrier()

          def chunk(s, _):
            pltpu.sync_copy(dst_hbm.at[r, tid, s], dstbuf)
            for q in range(SEG_B):
              pltpu.async_copy(onesbuf, acc.at[dstbuf.at[q]],
                               ssem_a, add=True)
            for q in range(SEG_B):
              pltpu.make_async_copy(onesbuf, acc.at[dstbuf.at[0]],
                                    ssem_a).wait()
            return 0

          lax.fori_loop(0, N_SEG, chunk, 0)
          plsc.subcore_barrier()
          pltpu.sync_copy(acc.at[pl.ds(rbase, STRIPE)],
                          deg_hbm.at[r, pl.ds(rbase, STRIPE)])

      return 0

    lax.fori_loop(0, PASS_PER_CORE, one_pass, 0)

  return k(xflat, sidx, dst4, zeros_in, ones_in)


def _tc_combine(agg, deg, W, b3):
  BN = 256
  grid = (N_PAD // BN,)

  def body(agg_ref, deg_ref, w_ref, b_ref, out_ref):
    acc = (b_ref[0, 0] + b_ref[1, 0] + b_ref[2, 0])[None, :]
    acc = jnp.broadcast_to(acc, (BN, FEAT))
    for r in range(N_REL):
      h = jax.lax.dot_general(
          agg_ref[r], w_ref[r], (((1,), (0,)), ((), ())),
          precision=jax.lax.Precision.HIGHEST,
          preferred_element_type=jnp.float32)
      d = deg_ref[r, :, 0]
      norm = jnp.where(d > 0.0, 1.0 / d, 0.0)
      acc = acc + h * norm[:, None]
    out_ref[...] = acc

  return pl.pallas_call(
      body,
      grid=grid,
      in_specs=[
          pl.BlockSpec((N_REL, BN, FEAT), lambda i: (0, i, 0)),
          pl.BlockSpec((N_REL, BN, LANES), lambda i: (0, i, 0)),
          pl.BlockSpec((N_REL, FEAT, FEAT), lambda i: (0, 0, 0)),
          pl.BlockSpec((N_REL, 1, FEAT), lambda i: (0, 0, 0)),
      ],
      out_specs=pl.BlockSpec((BN, FEAT), lambda i: (i, 0)),
      out_shape=jax.ShapeDtypeStruct((N_PAD, FEAT), jnp.float32),
  )(agg, deg, W, b3)


def kernel(x, edge_index, W, b):
  src = edge_index[:, 0, :].astype(jnp.int32)
  dst = edge_index[:, 1, :].astype(jnp.int32)
  pad = E_PAD - N_EDGES
  src = jnp.pad(src, ((0, 0), (0, pad)))
  dst = jnp.pad(dst, ((0, 0), (0, pad)), constant_values=N_NODES)

  c_ids = jnp.arange(CSLICES, dtype=jnp.int32)
  sidx = (src[:, None, :] * CSLICES + c_ids[None, :, None])
  sidx = sidx.reshape(N_REL, CSLICES, N_TILES, N_SEG, SEG_B * BATCH)
  dst4 = dst.reshape(N_REL, N_TILES, N_SEG, SEG_B, BATCH)
  xflat = x.reshape(CSLICES * N_NODES, LANES)

  zeros_in = jnp.zeros((ZROWS, LANES), jnp.float32)
  ones_in = jnp.ones((BATCH, LANES), jnp.float32)

  agg, deg = _sc_segment_sums(xflat, sidx, dst4, zeros_in, ones_in)
  out = _tc_combine(agg, deg, W, b.reshape(N_REL, 1, FEAT))
  return out[:N_NODES]
